# baseline (speedup 1.0000x reference)
import functools

import jax
import jax.numpy as jnp
from jax import lax
from jax.experimental import pallas as pl
from jax.experimental.pallas import tpu as pltpu
from jax.experimental.pallas import tpu_sc as plsc

K = 100000
KB = 5000
NB = K // KB
NQ = 16
D = 64


def _dist_body(mem_ref, q_ref, bidx_ref, minv_ref, bestv_scr, bidx_scr):
    i = pl.program_id(0)
    mem = mem_ref[...]
    q = q_ref[...]
    ones_w = jnp.ones((1, D), dtype=jnp.float32)
    norms = jax.lax.dot_general(
        mem * mem, ones_w, (((1,), (1,)), ((), ())),
        preferred_element_type=jnp.float32,
        precision=jax.lax.Precision.HIGHEST)
    dots = jax.lax.dot_general(
        mem, q, (((1,), (1,)), ((), ())),
        preferred_element_type=jnp.float32,
        precision=jax.lax.Precision.HIGHEST)
    dist = norms - 2.0 * dots
    bmin = jnp.min(dist, axis=0, keepdims=True)
    rows = jax.lax.broadcasted_iota(jnp.int32, (KB, NQ), 0) + i * KB
    bidx = jnp.min(jnp.where(dist == bmin, rows, K),
                   axis=0, keepdims=True)

    @pl.when(i == 0)
    def _init():
        bestv_scr[...] = bmin
        bidx_scr[...] = bidx

    @pl.when(i > 0)
    def _update():
        prev = bestv_scr[...]
        upd = bmin < prev
        bestv_scr[...] = jnp.where(upd, bmin, prev)
        bidx_scr[...] = jnp.where(upd, bidx, bidx_scr[...])

    @pl.when(i == NB - 1)
    def _final():
        bidx_ref[...] = bidx_scr[...]
        qnt = jax.lax.dot_general(
            ones_w, q * q, (((1,), (1,)), ((), ())),
            preferred_element_type=jnp.float32,
            precision=jax.lax.Precision.HIGHEST)
        minv_ref[...] = jnp.min(bestv_scr[...] + qnt).reshape(1, 1)


@functools.cache
def _make_sc_gather():
    mesh = plsc.VectorSubcoreMesh(core_axis_name="c", subcore_axis_name="s")

    @functools.partial(
        pl.kernel,
        mesh=mesh,
        out_type=jax.ShapeDtypeStruct((NQ, D), jnp.float32),
        scratch_types=[
            pltpu.VMEM((NQ,), jnp.int32),
            pltpu.VMEM((NQ, D), jnp.float32),
            pltpu.SemaphoreType.DMA,
        ],
        compiler_params=pltpu.CompilerParams(use_tc_tiling_on_sc=False),
    )
    def _sc_gather(idx_hbm, table_hbm, out_hbm, idx_v, rows_v, sem):
        wid = lax.axis_index("s") * 2 + lax.axis_index("c")

        @pl.when(wid == 0)
        def _():
            pltpu.sync_copy(idx_hbm, idx_v)
            pltpu.async_copy(table_hbm.at[idx_v], rows_v, sem).wait()
            pltpu.sync_copy(rows_v, out_hbm)

    return _sc_gather


def kernel(inpt, in_memory):
    bidx, minv = pl.pallas_call(
        _dist_body,
        grid=(NB,),
        in_specs=[
            pl.BlockSpec((KB, D), lambda i: (i, 0)),
            pl.BlockSpec((NQ, D), lambda i: (0, 0)),
        ],
        out_specs=[
            pl.BlockSpec((1, NQ), lambda i: (0, 0)),
            pl.BlockSpec((1, 1), lambda i: (0, 0)),
        ],
        out_shape=[
            jax.ShapeDtypeStruct((1, NQ), jnp.int32),
            jax.ShapeDtypeStruct((1, 1), jnp.float32),
        ],
        scratch_shapes=[
            pltpu.VMEM((1, NQ), jnp.float32),
            pltpu.VMEM((1, NQ), jnp.int32),
        ],
        compiler_params=pltpu.CompilerParams(
            dimension_semantics=("arbitrary",)),
    )(in_memory, inpt)
    matched = _make_sc_gather()(bidx.reshape(NQ), in_memory)
    return matched, minv[0, 0]


# speedup vs baseline: 1.1461x; 1.1461x over previous
import functools

import jax
import jax.numpy as jnp
from jax import lax
from jax.experimental import pallas as pl
from jax.experimental.pallas import tpu as pltpu
from jax.experimental.pallas import tpu_sc as plsc

K = 100000
NQ = 16
D = 64
G = 8
DP = G * D
KR = K // G
KB8 = 1256
NB = -(-KR // KB8)
BIGF = 3.0e38


def _dist_body(mem_ref, q_ref, bidx_ref, minv_ref,
               wq_scr, wn_scr, bestv_scr, bidx_scr):
    i = pl.program_id(0)

    @pl.when(i == 0)
    def _build_weights():
        q = q_ref[...]
        ident = (jax.lax.broadcasted_iota(jnp.int32, (D, D), 0) ==
                 jax.lax.broadcasted_iota(jnp.int32, (D, D), 1)
                 ).astype(jnp.float32)
        qt = jax.lax.dot_general(
            ident, q, (((1,), (1,)), ((), ())),
            preferred_element_type=jnp.float32,
            precision=jax.lax.Precision.HIGHEST)
        qt_tall = jnp.concatenate([qt] * G, axis=0)
        qt_wide = jnp.concatenate([qt_tall] * G, axis=1)
        rgrp = jax.lax.broadcasted_iota(jnp.int32, (DP, G * NQ), 0) // D
        cgrp = jax.lax.broadcasted_iota(jnp.int32, (DP, G * NQ), 1) // NQ
        gmask = (rgrp == cgrp).astype(jnp.float32)
        wq_scr[...] = -2.0 * qt_wide * gmask
        wn_scr[...] = gmask

    mem = mem_ref[...]
    dist = (
        jax.lax.dot_general(
            mem, wq_scr[...], (((1,), (0,)), ((), ())),
            preferred_element_type=jnp.float32,
            precision=jax.lax.Precision.HIGHEST)
        + jax.lax.dot_general(
            mem * mem, wn_scr[...], (((1,), (0,)), ((), ())),
            preferred_element_type=jnp.float32,
            precision=jax.lax.Precision.HIGHEST)
    )

    kidx = ((jax.lax.broadcasted_iota(jnp.int32, (KB8, G * NQ), 0)
             + i * KB8) * G
            + jax.lax.broadcasted_iota(jnp.int32, (KB8, G * NQ), 1) // NQ)
    dist = jnp.where(kidx < K, dist, BIGF)
    bmin = jnp.min(dist, axis=0, keepdims=True)
    bidx = jnp.min(jnp.where(dist == bmin, kidx, K),
                   axis=0, keepdims=True)

    @pl.when(i == 0)
    def _init():
        bestv_scr[...] = bmin
        bidx_scr[...] = bidx

    @pl.when(i > 0)
    def _update():
        prev = bestv_scr[...]
        upd = bmin < prev
        bestv_scr[...] = jnp.where(upd, bmin, prev)
        bidx_scr[...] = jnp.where(upd, bidx, bidx_scr[...])

    @pl.when(i == NB - 1)
    def _final():
        v = bestv_scr[...]
        ix = bidx_scr[...]
        for s in (4 * NQ, 2 * NQ, NQ):
            vs = jnp.concatenate(
                [v[:, s:], jnp.full((1, s), BIGF, jnp.float32)], axis=1)
            ixs = jnp.concatenate(
                [ix[:, s:], jnp.full((1, s), K, jnp.int32)], axis=1)
            take = (vs < v) | ((vs == v) & (ixs < ix))
            v = jnp.where(take, vs, v)
            ix = jnp.where(take, ixs, ix)
        bidx_ref[...] = ix[:, :NQ]
        q = q_ref[...]
        qnt = jax.lax.dot_general(
            jnp.ones((1, D), jnp.float32), q * q, (((1,), (1,)), ((), ())),
            preferred_element_type=jnp.float32,
            precision=jax.lax.Precision.HIGHEST)
        minv_ref[...] = jnp.min(v[:, :NQ] + qnt).reshape(1, 1)


@functools.cache
def _make_sc_gather():
    mesh = plsc.VectorSubcoreMesh(core_axis_name="c", subcore_axis_name="s")

    @functools.partial(
        pl.kernel,
        mesh=mesh,
        out_type=jax.ShapeDtypeStruct((NQ, D), jnp.float32),
        scratch_types=[
            pltpu.VMEM((NQ,), jnp.int32),
            pltpu.VMEM((NQ, D), jnp.float32),
            pltpu.SemaphoreType.DMA,
        ],
        compiler_params=pltpu.CompilerParams(use_tc_tiling_on_sc=False),
    )
    def _sc_gather(idx_hbm, table_hbm, out_hbm, idx_v, rows_v, sem):
        wid = lax.axis_index("s") * 2 + lax.axis_index("c")

        @pl.when(wid == 0)
        def _():
            pltpu.sync_copy(idx_hbm, idx_v)
            pltpu.async_copy(table_hbm.at[idx_v], rows_v, sem).wait()
            pltpu.sync_copy(rows_v, out_hbm)

    return _sc_gather


def kernel(inpt, in_memory):
    mem_packed = in_memory.reshape(KR, DP)
    bidx, minv = pl.pallas_call(
        _dist_body,
        grid=(NB,),
        in_specs=[
            pl.BlockSpec((KB8, DP), lambda i: (i, 0)),
            pl.BlockSpec((NQ, D), lambda i: (0, 0)),
        ],
        out_specs=[
            pl.BlockSpec((1, NQ), lambda i: (0, 0)),
            pl.BlockSpec((1, 1), lambda i: (0, 0)),
        ],
        out_shape=[
            jax.ShapeDtypeStruct((1, NQ), jnp.int32),
            jax.ShapeDtypeStruct((1, 1), jnp.float32),
        ],
        scratch_shapes=[
            pltpu.VMEM((DP, G * NQ), jnp.float32),
            pltpu.VMEM((DP, G * NQ), jnp.float32),
            pltpu.VMEM((1, G * NQ), jnp.float32),
            pltpu.VMEM((1, G * NQ), jnp.int32),
        ],
        compiler_params=pltpu.CompilerParams(
            dimension_semantics=("arbitrary",)),
    )(mem_packed, inpt)
    matched = _make_sc_gather()(bidx.reshape(NQ), in_memory)
    return matched, minv[0, 0]
